# baseline (speedup 1.0000x reference)
import functools
import math

import numpy as np
import jax
import jax.numpy as jnp
from jax.experimental import pallas as pl
from jax.experimental.pallas import tpu as pltpu

_WS = 46
_FACTOR = 2
_NEG_INF = float("-inf")


def _rpi_slice(n_rows, n_cols):
    ws = _WS
    idx = np.arange(ws * ws)
    r, c = idx // ws, idx % ws
    ar, ac = r[:n_rows, None], c[:n_rows, None]
    br, bc = r[None, :n_cols], c[None, :n_cols]
    return ((ar - br + ws - 1) * (2 * ws - 1) + (ac - bc + ws - 1)).astype(np.int32)


def _body(q_ref, k_ref, kf_ref, v_ref, bias_ref, ctx_ref, attn_ref,
          *, u, d, q_chunk, c_chunk, hb):
    L = q_ref.shape[1]
    scale = 1.0 / math.sqrt(d)

    W = hb * d
    tri = (jax.lax.broadcasted_iota(jnp.int32, (c_chunk, c_chunk), 0)
           >= jax.lax.broadcasted_iota(jnp.int32, (c_chunk, c_chunk), 1)
           ).astype(jnp.float32)
    iota = jax.lax.broadcasted_iota(jnp.int32, (1, L), 1)
    lane = jax.lax.broadcasted_iota(jnp.int32, (1, W), 1)
    bias = bias_ref[...]

    carry = jnp.zeros((1, W), jnp.float32)
    for i in range(L // c_chunk):
        vc = v_ref[0, i * c_chunk:(i + 1) * c_chunk, :]
        pc = jnp.dot(tri, vc, preferred_element_type=jnp.float32,
                     precision=jax.lax.Precision.HIGHEST) + carry
        ctx_ref[0, i * c_chunk:(i + 1) * c_chunk, :] = pc
        carry = pc[c_chunk - 1:c_chunk, :]

    Ms = []
    for h in range(hb):
        lo, hi = h * d, (h + 1) * d
        Q = q_ref[0, :, lo:hi]
        Ks = kf_ref[0, :, lo:hi]
        m_parts = []
        for i in range(L // q_chunk):
            qc = Q[i * q_chunk:(i + 1) * q_chunk]
            s = jax.lax.dot_general(qc, Ks, (((1,), (1,)), ((), ())),
                                    preferred_element_type=jnp.float32)
            m_parts.append(jnp.max(s, axis=1, keepdims=True)
                           - jnp.sum(s, axis=1, keepdims=True) * (1.0 / L))
        Ms.append(jnp.transpose(jnp.concatenate(m_parts, axis=0)))

    all_idxs, all_upd = [], []
    for h in range(hb):
        lo, hi = h * d, (h + 1) * d
        idxs = []
        Mw = Ms[h]
        for _ in range(u):
            mval = jnp.max(Mw)
            idx = jnp.min(jnp.where(Mw >= mval, iota, L))
            idxs.append(idx)
            Mw = jnp.where(iota == idx, _NEG_INF, Mw)
        all_idxs.append(idxs)

        Qr = jnp.concatenate(
            [q_ref[0, pl.ds(idxs[i], 1), :] for i in range(u)], axis=0)[:, lo:hi]

        K = k_ref[0, :, lo:hi]
        scores = jax.lax.dot_general(Qr, K, (((1,), (1,)), ((), ())),
                                     preferred_element_type=jnp.float32)
        scores = (scores + bias) * scale
        smax = jnp.max(scores, axis=1, keepdims=True)
        e = jnp.exp(scores - smax)
        attn = e / jnp.sum(e, axis=1, keepdims=True)
        attn_ref[0, h] = attn
        V = v_ref[0, :, lo:hi]
        update = jnp.dot(attn, V, preferred_element_type=jnp.float32)

        parts = []
        if lo > 0:
            parts.append(jnp.zeros((u, lo), jnp.float32))
        parts.append(update)
        if W > hi:
            parts.append(jnp.zeros((u, W - hi), jnp.float32))
        all_upd.append(jnp.concatenate(parts, axis=1) if len(parts) > 1
                       else update)

    for h in range(hb):
        lo, hi = h * d, (h + 1) * d
        hmask = (lane >= lo) & (lane < hi)
        idxs, upd_full = all_idxs[h], all_upd[h]
        for i in range(u):
            row = ctx_ref[0, pl.ds(idxs[i], 1), :]
            ctx_ref[0, pl.ds(idxs[i], 1), :] = jnp.where(
                hmask, upd_full[i:i + 1, :], row)


def kernel(queries, keys, values, attn_mask, rel_pos_bias_table):
    del attn_mask
    B, L, H, D = queries.shape
    u = min(_FACTOR * int(np.ceil(np.log(L))), L)
    hb = 4 if H % 4 == 0 else H

    q2 = queries.reshape(B, L, H * D)
    k2 = keys.reshape(B, L, H * D)
    v2 = values.reshape(B, L, H * D)
    kf = keys.reshape(B, L // 2, 2 * H * D)
    bias = rel_pos_bias_table[jnp.asarray(_rpi_slice(u, L)), 0]

    body = functools.partial(_body, u=u, d=D, q_chunk=min(512, L),
                             c_chunk=min(128, L), hb=hb)
    W = hb * D
    ctx, attn = pl.pallas_call(
        body,
        grid=(B, H // hb),
        in_specs=[
            pl.BlockSpec((1, L, W), lambda b, g: (b, 0, g)),
            pl.BlockSpec((1, L, W), lambda b, g: (b, 0, g)),
            pl.BlockSpec((1, L // 2, W), lambda b, g: (b, 0, g)),
            pl.BlockSpec((1, L, W), lambda b, g: (b, 0, g)),
            pl.BlockSpec((u, L), lambda b, g: (0, 0)),
        ],
        out_specs=[
            pl.BlockSpec((1, L, W), lambda b, g: (b, 0, g)),
            pl.BlockSpec((1, hb, u, L), lambda b, g: (b, g, 0, 0)),
        ],
        out_shape=[
            jax.ShapeDtypeStruct((B, L, H * D), jnp.float32),
            jax.ShapeDtypeStruct((B, H, u, L), jnp.float32),
        ],
        compiler_params=pltpu.CompilerParams(
            dimension_semantics=("parallel", "parallel")),
    )(q2, k2, kf, v2, bias)
    return ctx.reshape(B, L, H, D), attn


# speedup vs baseline: 1.2213x; 1.2213x over previous
import functools
import math

import numpy as np
import jax
import jax.numpy as jnp
from jax.experimental import pallas as pl
from jax.experimental.pallas import tpu as pltpu

_WS = 46
_FACTOR = 2
_NEG_INF = float("-inf")


def _rpi_slice(n_rows, n_cols):
    ws = _WS
    idx = np.arange(ws * ws)
    r, c = idx // ws, idx % ws
    ar, ac = r[:n_rows, None], c[:n_rows, None]
    br, bc = r[None, :n_cols], c[None, :n_cols]
    return ((ar - br + ws - 1) * (2 * ws - 1) + (ac - bc + ws - 1)).astype(np.int32)


def _body(q_ref, k_ref, v_ref, bias_ref, ctx_ref, attn_ref,
          *, u, d, q_chunk, c_chunk, hb):
    L = q_ref.shape[1]
    scale = 1.0 / math.sqrt(d)

    W = hb * d
    tri = (jax.lax.broadcasted_iota(jnp.int32, (c_chunk, c_chunk), 0)
           >= jax.lax.broadcasted_iota(jnp.int32, (c_chunk, c_chunk), 1)
           ).astype(jnp.float32)
    iota = jax.lax.broadcasted_iota(jnp.int32, (1, L), 1)
    lane = jax.lax.broadcasted_iota(jnp.int32, (1, W), 1)
    bias = bias_ref[...]

    carry = jnp.zeros((1, W), jnp.float32)
    for i in range(L // c_chunk):
        vc = v_ref[0, i * c_chunk:(i + 1) * c_chunk, :]
        pc = jnp.dot(tri, vc, preferred_element_type=jnp.float32,
                     precision=jax.lax.Precision.HIGHEST) + carry
        ctx_ref[0, i * c_chunk:(i + 1) * c_chunk, :] = pc
        carry = pc[c_chunk - 1:c_chunk, :]

    Ms = []
    for h in range(hb):
        lo, hi = h * d, (h + 1) * d
        Q = q_ref[0, :, lo:hi]
        Kw = k_ref[0, :, lo:hi]
        Ks = Kw.reshape(Kw.shape[0] // 2, 2, Kw.shape[1])[:, 0, :]
        m_parts = []
        for i in range(L // q_chunk):
            qc = Q[i * q_chunk:(i + 1) * q_chunk]
            s = jax.lax.dot_general(qc, Ks, (((1,), (1,)), ((), ())),
                                    preferred_element_type=jnp.float32)
            m_parts.append(jnp.max(s, axis=1, keepdims=True)
                           - jnp.sum(s, axis=1, keepdims=True) * (1.0 / L))
        Ms.append(jnp.transpose(jnp.concatenate(m_parts, axis=0)))

    all_idxs, all_upd = [], []
    for h in range(hb):
        lo, hi = h * d, (h + 1) * d
        idxs = []
        Mw = Ms[h]
        for _ in range(u):
            mval = jnp.max(Mw)
            idx = jnp.min(jnp.where(Mw >= mval, iota, L))
            idxs.append(idx)
            Mw = jnp.where(iota == idx, _NEG_INF, Mw)
        all_idxs.append(idxs)

        Qr = jnp.concatenate(
            [q_ref[0, pl.ds(idxs[i], 1), :] for i in range(u)], axis=0)[:, lo:hi]

        K = k_ref[0, :, lo:hi]
        scores = jax.lax.dot_general(Qr, K, (((1,), (1,)), ((), ())),
                                     preferred_element_type=jnp.float32)
        scores = (scores + bias) * scale
        smax = jnp.max(scores, axis=1, keepdims=True)
        e = jnp.exp(scores - smax)
        attn = e / jnp.sum(e, axis=1, keepdims=True)
        attn_ref[0, h] = attn
        V = v_ref[0, :, lo:hi]
        update = jnp.dot(attn, V, preferred_element_type=jnp.float32)

        parts = []
        if lo > 0:
            parts.append(jnp.zeros((u, lo), jnp.float32))
        parts.append(update)
        if W > hi:
            parts.append(jnp.zeros((u, W - hi), jnp.float32))
        all_upd.append(jnp.concatenate(parts, axis=1) if len(parts) > 1
                       else update)

    for h in range(hb):
        lo, hi = h * d, (h + 1) * d
        hmask = (lane >= lo) & (lane < hi)
        idxs, upd_full = all_idxs[h], all_upd[h]
        for i in range(u):
            row = ctx_ref[0, pl.ds(idxs[i], 1), :]
            ctx_ref[0, pl.ds(idxs[i], 1), :] = jnp.where(
                hmask, upd_full[i:i + 1, :], row)


def kernel(queries, keys, values, attn_mask, rel_pos_bias_table):
    del attn_mask
    B, L, H, D = queries.shape
    u = min(_FACTOR * int(np.ceil(np.log(L))), L)
    hb = 4 if H % 4 == 0 else H

    q2 = queries.reshape(B, L, H * D)
    k2 = keys.reshape(B, L, H * D)
    v2 = values.reshape(B, L, H * D)
    bias = rel_pos_bias_table[jnp.asarray(_rpi_slice(u, L)), 0]

    body = functools.partial(_body, u=u, d=D, q_chunk=min(512, L),
                             c_chunk=min(128, L), hb=hb)
    W = hb * D
    ctx, attn = pl.pallas_call(
        body,
        grid=(B, H // hb),
        in_specs=[
            pl.BlockSpec((1, L, W), lambda b, g: (b, 0, g)),
            pl.BlockSpec((1, L, W), lambda b, g: (b, 0, g)),
            pl.BlockSpec((1, L, W), lambda b, g: (b, 0, g)),
            pl.BlockSpec((u, L), lambda b, g: (0, 0)),
        ],
        out_specs=[
            pl.BlockSpec((1, L, W), lambda b, g: (b, 0, g)),
            pl.BlockSpec((1, hb, u, L), lambda b, g: (b, g, 0, 0)),
        ],
        out_shape=[
            jax.ShapeDtypeStruct((B, L, H * D), jnp.float32),
            jax.ShapeDtypeStruct((B, H, u, L), jnp.float32),
        ],
        compiler_params=pltpu.CompilerParams(
            dimension_semantics=("parallel", "parallel")),
    )(q2, k2, v2, bias)
    return ctx.reshape(B, L, H, D), attn


# speedup vs baseline: 1.8416x; 1.5079x over previous
import functools
import math

import numpy as np
import jax
import jax.numpy as jnp
from jax import lax
from jax.experimental import pallas as pl
from jax.experimental.pallas import tpu as pltpu
from jax.experimental.pallas import tpu_sc as plsc

_WS = 46
_FACTOR = 2
_NEG_INF = float("-inf")


def _rpi_slice(n_rows, n_cols):
    ws = _WS
    idx = np.arange(ws * ws)
    r, c = idx // ws, idx % ws
    ar, ac = r[:n_rows, None], c[:n_rows, None]
    br, bc = r[None, :n_cols], c[None, :n_cols]
    return ((ar - br + ws - 1) * (2 * ws - 1) + (ac - bc + ws - 1)).astype(np.int32)


def _bias_gather_sc(table_flat, idx_flat):
    n_idx = idx_flat.shape[0]
    n_tab = table_flat.shape[0]
    info = plsc.get_sparse_core_info()
    nw = info.num_cores * info.num_subcores
    lanes = info.num_lanes
    per_w = n_idx // nw
    mesh = plsc.VectorSubcoreMesh(core_axis_name="c", subcore_axis_name="s")

    @functools.partial(
        pl.kernel, mesh=mesh,
        out_type=jax.ShapeDtypeStruct((n_idx,), jnp.float32),
        compiler_params=pltpu.CompilerParams(needs_layout_passes=False),
        scratch_types=[
            pltpu.VMEM((n_tab,), jnp.float32),
            pltpu.VMEM((per_w,), jnp.int32),
            pltpu.VMEM((per_w,), jnp.float32),
        ],
    )
    def k(tab_hbm, idx_hbm, out_hbm, tab_v, idx_v, val_v):
        wid = lax.axis_index("s") * info.num_cores + lax.axis_index("c")
        base = wid * per_w
        pltpu.sync_copy(tab_hbm, tab_v)
        pltpu.sync_copy(idx_hbm.at[pl.ds(base, per_w)], idx_v)
        for j in range(per_w // lanes):
            iv = idx_v[pl.ds(j * lanes, lanes)]
            val_v[pl.ds(j * lanes, lanes)] = plsc.load_gather(tab_v, [iv])
        pltpu.sync_copy(val_v, out_hbm.at[pl.ds(base, per_w)])

    return k(table_flat, idx_flat)


def _body(q_ref, k_ref, v_ref, bias_ref, ctx_ref, attn_ref,
          *, u, d, q_chunk, c_chunk, hb):
    L = q_ref.shape[1]
    scale = 1.0 / math.sqrt(d)

    W = hb * d
    tri = (jax.lax.broadcasted_iota(jnp.int32, (c_chunk, c_chunk), 0)
           >= jax.lax.broadcasted_iota(jnp.int32, (c_chunk, c_chunk), 1)
           ).astype(jnp.float32)
    iota = jax.lax.broadcasted_iota(jnp.int32, (1, L), 1)
    lane = jax.lax.broadcasted_iota(jnp.int32, (1, W), 1)
    bias = bias_ref[...]

    carry = jnp.zeros((1, W), jnp.float32)
    for i in range(L // c_chunk):
        vc = v_ref[0, i * c_chunk:(i + 1) * c_chunk, :]
        pc = jnp.dot(tri, vc, preferred_element_type=jnp.float32,
                     precision=jax.lax.Precision.HIGHEST) + carry
        ctx_ref[0, i * c_chunk:(i + 1) * c_chunk, :] = pc
        carry = pc[c_chunk - 1:c_chunk, :]

    Ms = []
    for h in range(hb):
        lo, hi = h * d, (h + 1) * d
        Q = q_ref[0, :, lo:hi]
        Kw = k_ref[0, :, lo:hi]
        Ks = Kw.reshape(Kw.shape[0] // 2, 2, Kw.shape[1])[:, 0, :]
        m_parts = []
        for i in range(L // q_chunk):
            qc = Q[i * q_chunk:(i + 1) * q_chunk]
            s = jax.lax.dot_general(qc, Ks, (((1,), (1,)), ((), ())),
                                    preferred_element_type=jnp.float32)
            m_parts.append(jnp.max(s, axis=1, keepdims=True)
                           - jnp.sum(s, axis=1, keepdims=True) * (1.0 / L))
        Ms.append(jnp.transpose(jnp.concatenate(m_parts, axis=0)))

    all_idxs, all_upd = [], []
    for h in range(hb):
        lo, hi = h * d, (h + 1) * d
        idxs = []
        Mw = Ms[h]
        for _ in range(u):
            mval = jnp.max(Mw)
            idx = jnp.min(jnp.where(Mw >= mval, iota, L))
            idxs.append(idx)
            Mw = jnp.where(iota == idx, _NEG_INF, Mw)
        all_idxs.append(idxs)

        Qr = jnp.concatenate(
            [q_ref[0, pl.ds(idxs[i], 1), :] for i in range(u)], axis=0)[:, lo:hi]

        K = k_ref[0, :, lo:hi]
        scores = jax.lax.dot_general(Qr, K, (((1,), (1,)), ((), ())),
                                     preferred_element_type=jnp.float32)
        scores = (scores + bias) * scale
        smax = jnp.max(scores, axis=1, keepdims=True)
        e = jnp.exp(scores - smax)
        attn = e / jnp.sum(e, axis=1, keepdims=True)
        attn_ref[0, h] = attn
        V = v_ref[0, :, lo:hi]
        update = jnp.dot(attn, V, preferred_element_type=jnp.float32)

        parts = []
        if lo > 0:
            parts.append(jnp.zeros((u, lo), jnp.float32))
        parts.append(update)
        if W > hi:
            parts.append(jnp.zeros((u, W - hi), jnp.float32))
        all_upd.append(jnp.concatenate(parts, axis=1) if len(parts) > 1
                       else update)

    for h in range(hb):
        lo, hi = h * d, (h + 1) * d
        hmask = (lane >= lo) & (lane < hi)
        idxs, upd_full = all_idxs[h], all_upd[h]
        for i in range(u):
            row = ctx_ref[0, pl.ds(idxs[i], 1), :]
            ctx_ref[0, pl.ds(idxs[i], 1), :] = jnp.where(
                hmask, upd_full[i:i + 1, :], row)


def kernel(queries, keys, values, attn_mask, rel_pos_bias_table):
    del attn_mask
    B, L, H, D = queries.shape
    u = min(_FACTOR * int(np.ceil(np.log(L))), L)
    hb = 4 if H % 4 == 0 else H

    q2 = queries.reshape(B, L, H * D)
    k2 = keys.reshape(B, L, H * D)
    v2 = values.reshape(B, L, H * D)
    idx_flat = jnp.asarray(_rpi_slice(u, L).reshape(-1))
    tab_flat = rel_pos_bias_table[:, 0]
    pad = (-tab_flat.shape[0]) % 128
    tab_flat = jnp.pad(tab_flat, (0, pad))
    bias = _bias_gather_sc(tab_flat, idx_flat).reshape(u, L)

    body = functools.partial(_body, u=u, d=D, q_chunk=min(512, L),
                             c_chunk=min(128, L), hb=hb)
    W = hb * D
    ctx, attn = pl.pallas_call(
        body,
        grid=(B, H // hb),
        in_specs=[
            pl.BlockSpec((1, L, W), lambda b, g: (b, 0, g)),
            pl.BlockSpec((1, L, W), lambda b, g: (b, 0, g)),
            pl.BlockSpec((1, L, W), lambda b, g: (b, 0, g)),
            pl.BlockSpec((u, L), lambda b, g: (0, 0)),
        ],
        out_specs=[
            pl.BlockSpec((1, L, W), lambda b, g: (b, 0, g)),
            pl.BlockSpec((1, hb, u, L), lambda b, g: (b, g, 0, 0)),
        ],
        out_shape=[
            jax.ShapeDtypeStruct((B, L, H * D), jnp.float32),
            jax.ShapeDtypeStruct((B, H, u, L), jnp.float32),
        ],
        compiler_params=pltpu.CompilerParams(
            dimension_semantics=("parallel", "parallel")),
    )(q2, k2, v2, bias)
    return ctx.reshape(B, L, H, D), attn


# speedup vs baseline: 1.8549x; 1.0072x over previous
import functools
import math

import numpy as np
import jax
import jax.numpy as jnp
from jax import lax
from jax.experimental import pallas as pl
from jax.experimental.pallas import tpu as pltpu
from jax.experimental.pallas import tpu_sc as plsc

_WS = 46
_FACTOR = 2
_NEG_INF = float("-inf")


def _rpi_slice(n_rows, n_cols):
    ws = _WS
    idx = np.arange(ws * ws)
    r, c = idx // ws, idx % ws
    ar, ac = r[:n_rows, None], c[:n_rows, None]
    br, bc = r[None, :n_cols], c[None, :n_cols]
    return ((ar - br + ws - 1) * (2 * ws - 1) + (ac - bc + ws - 1)).astype(np.int32)


def _bias_gather_sc(table_flat, idx_flat):
    n_idx = idx_flat.shape[0]
    n_tab = table_flat.shape[0]
    info = plsc.get_sparse_core_info()
    nw = info.num_cores * info.num_subcores
    lanes = info.num_lanes
    per_w = n_idx // nw
    mesh = plsc.VectorSubcoreMesh(core_axis_name="c", subcore_axis_name="s")

    @functools.partial(
        pl.kernel, mesh=mesh,
        out_type=jax.ShapeDtypeStruct((n_idx,), jnp.float32),
        compiler_params=pltpu.CompilerParams(needs_layout_passes=False),
        scratch_types=[
            pltpu.VMEM((n_tab,), jnp.float32),
            pltpu.VMEM((per_w,), jnp.int32),
            pltpu.VMEM((per_w,), jnp.float32),
        ],
    )
    def k(tab_hbm, idx_hbm, out_hbm, tab_v, idx_v, val_v):
        wid = lax.axis_index("s") * info.num_cores + lax.axis_index("c")
        base = wid * per_w
        pltpu.sync_copy(tab_hbm, tab_v)
        pltpu.sync_copy(idx_hbm.at[pl.ds(base, per_w)], idx_v)
        for j in range(per_w // lanes):
            iv = idx_v[pl.ds(j * lanes, lanes)]
            val_v[pl.ds(j * lanes, lanes)] = plsc.load_gather(tab_v, [iv])
        pltpu.sync_copy(val_v, out_hbm.at[pl.ds(base, per_w)])

    return k(table_flat, idx_flat)


def _body(q_ref, k_ref, v_ref, bias_ref, ctx_ref, attn_ref,
          *, u, d, q_chunk, c_chunk, hb):
    L = q_ref.shape[1]
    scale = 1.0 / math.sqrt(d)

    W = hb * d
    tri = (jax.lax.broadcasted_iota(jnp.int32, (c_chunk, c_chunk), 0)
           >= jax.lax.broadcasted_iota(jnp.int32, (c_chunk, c_chunk), 1)
           ).astype(jnp.float32)
    iota = jax.lax.broadcasted_iota(jnp.int32, (1, L), 1)
    lane = jax.lax.broadcasted_iota(jnp.int32, (1, W), 1)
    bias = bias_ref[...]

    carry = jnp.zeros((1, W), jnp.float32)
    for i in range(L // c_chunk):
        vc = v_ref[0, i * c_chunk:(i + 1) * c_chunk, :]
        pc = jnp.dot(tri, vc, preferred_element_type=jnp.float32,
                     precision=jax.lax.Precision.HIGHEST) + carry
        ctx_ref[0, i * c_chunk:(i + 1) * c_chunk, :] = pc
        carry = pc[c_chunk - 1:c_chunk, :]

    Ms = []
    for h in range(hb):
        lo, hi = h * d, (h + 1) * d
        Q = q_ref[0, :, lo:hi]
        Kw = k_ref[0, :, lo:hi]
        Ks = Kw.reshape(Kw.shape[0] // 2, 2, Kw.shape[1])[:, 0, :]
        m_parts = []
        for i in range(L // q_chunk):
            qc = Q[i * q_chunk:(i + 1) * q_chunk]
            s = jax.lax.dot_general(qc, Ks, (((1,), (1,)), ((), ())),
                                    preferred_element_type=jnp.float32)
            m_parts.append(jnp.max(s, axis=1, keepdims=True)
                           - jnp.sum(s, axis=1, keepdims=True) * (1.0 / L))
        Ms.append(jnp.transpose(jnp.concatenate(m_parts, axis=0)))

    all_idxs, all_upd = [], []
    for h in range(hb):
        lo, hi = h * d, (h + 1) * d
        idxs = []
        Mw = Ms[h]
        for _ in range(u):
            mval = jnp.max(Mw)
            idx = jnp.min(jnp.where(Mw >= mval, iota, L))
            idxs.append(idx)
            Mw = jnp.where(iota == idx, _NEG_INF, Mw)
        all_idxs.append(idxs)

        Qr = jnp.concatenate(
            [q_ref[0, pl.ds(idxs[i], 1), :] for i in range(u)], axis=0)[:, lo:hi]

        K = k_ref[0, :, lo:hi]
        scores = jax.lax.dot_general(Qr, K, (((1,), (1,)), ((), ())),
                                     preferred_element_type=jnp.float32)
        scores = (scores + bias) * scale
        smax = jnp.max(scores, axis=1, keepdims=True)
        e = jnp.exp(scores - smax)
        attn = e / jnp.sum(e, axis=1, keepdims=True)
        attn_ref[0, h] = attn
        V = v_ref[0, :, lo:hi]
        update = jnp.dot(attn, V, preferred_element_type=jnp.float32)

        parts = []
        if lo > 0:
            parts.append(jnp.zeros((u, lo), jnp.float32))
        parts.append(update)
        if W > hi:
            parts.append(jnp.zeros((u, W - hi), jnp.float32))
        all_upd.append(jnp.concatenate(parts, axis=1) if len(parts) > 1
                       else update)

    for h in range(hb):
        lo, hi = h * d, (h + 1) * d
        hmask = (lane >= lo) & (lane < hi)
        idxs, upd_full = all_idxs[h], all_upd[h]
        for i in range(u):
            row = ctx_ref[0, pl.ds(idxs[i], 1), :]
            ctx_ref[0, pl.ds(idxs[i], 1), :] = jnp.where(
                hmask, upd_full[i:i + 1, :], row)


def kernel(queries, keys, values, attn_mask, rel_pos_bias_table):
    del attn_mask
    B, L, H, D = queries.shape
    u = min(_FACTOR * int(np.ceil(np.log(L))), L)
    hb = 8 if H % 8 == 0 else H

    q2 = queries.reshape(B, L, H * D)
    k2 = keys.reshape(B, L, H * D)
    v2 = values.reshape(B, L, H * D)
    idx_flat = jnp.asarray(_rpi_slice(u, L).reshape(-1))
    tab_flat = rel_pos_bias_table[:, 0]
    pad = (-tab_flat.shape[0]) % 128
    tab_flat = jnp.pad(tab_flat, (0, pad))
    bias = _bias_gather_sc(tab_flat, idx_flat).reshape(u, L)

    body = functools.partial(_body, u=u, d=D, q_chunk=min(512, L),
                             c_chunk=min(128, L), hb=hb)
    W = hb * D
    ctx, attn = pl.pallas_call(
        body,
        grid=(B, H // hb),
        in_specs=[
            pl.BlockSpec((1, L, W), lambda b, g: (b, 0, g)),
            pl.BlockSpec((1, L, W), lambda b, g: (b, 0, g)),
            pl.BlockSpec((1, L, W), lambda b, g: (b, 0, g)),
            pl.BlockSpec((u, L), lambda b, g: (0, 0)),
        ],
        out_specs=[
            pl.BlockSpec((1, L, W), lambda b, g: (b, 0, g)),
            pl.BlockSpec((1, hb, u, L), lambda b, g: (b, g, 0, 0)),
        ],
        out_shape=[
            jax.ShapeDtypeStruct((B, L, H * D), jnp.float32),
            jax.ShapeDtypeStruct((B, H, u, L), jnp.float32),
        ],
        compiler_params=pltpu.CompilerParams(
            dimension_semantics=("parallel", "parallel")),
    )(q2, k2, v2, bias)
    return ctx.reshape(B, L, H, D), attn


# speedup vs baseline: 1.8839x; 1.0156x over previous
import functools
import math

import numpy as np
import jax
import jax.numpy as jnp
from jax import lax
from jax.experimental import pallas as pl
from jax.experimental.pallas import tpu as pltpu
from jax.experimental.pallas import tpu_sc as plsc

_WS = 46
_FACTOR = 2
_NEG_INF = float("-inf")


def _rpi_slice(n_rows, n_cols):
    ws = _WS
    idx = np.arange(ws * ws)
    r, c = idx // ws, idx % ws
    ar, ac = r[:n_rows, None], c[:n_rows, None]
    br, bc = r[None, :n_cols], c[None, :n_cols]
    return ((ar - br + ws - 1) * (2 * ws - 1) + (ac - bc + ws - 1)).astype(np.int32)


def _bias_gather_sc(table_flat, idx_flat):
    n_idx = idx_flat.shape[0]
    n_tab = table_flat.shape[0]
    info = plsc.get_sparse_core_info()
    nw = info.num_cores * info.num_subcores
    lanes = info.num_lanes
    per_w = n_idx // nw
    mesh = plsc.VectorSubcoreMesh(core_axis_name="c", subcore_axis_name="s")

    @functools.partial(
        pl.kernel, mesh=mesh,
        out_type=jax.ShapeDtypeStruct((n_idx,), jnp.float32),
        compiler_params=pltpu.CompilerParams(needs_layout_passes=False),
        scratch_types=[
            pltpu.VMEM((n_tab,), jnp.float32),
            pltpu.VMEM((per_w,), jnp.int32),
            pltpu.VMEM((per_w,), jnp.float32),
        ],
    )
    def k(tab_hbm, idx_hbm, out_hbm, tab_v, idx_v, val_v):
        wid = lax.axis_index("s") * info.num_cores + lax.axis_index("c")
        base = wid * per_w
        pltpu.sync_copy(tab_hbm, tab_v)
        pltpu.sync_copy(idx_hbm.at[pl.ds(base, per_w)], idx_v)
        for j in range(per_w // lanes):
            iv = idx_v[pl.ds(j * lanes, lanes)]
            val_v[pl.ds(j * lanes, lanes)] = plsc.load_gather(tab_v, [iv])
        pltpu.sync_copy(val_v, out_hbm.at[pl.ds(base, per_w)])

    return k(table_flat, idx_flat)


def _body(q_ref, k_ref, v_ref, bias_ref, ctx_ref, attn_ref,
          *, u, d, q_chunk, c_chunk, hb):
    L = q_ref.shape[1]
    scale = 1.0 / math.sqrt(d)

    W = hb * d
    tri = (jax.lax.broadcasted_iota(jnp.int32, (c_chunk, c_chunk), 0)
           >= jax.lax.broadcasted_iota(jnp.int32, (c_chunk, c_chunk), 1)
           ).astype(jnp.float32)
    iota = jax.lax.broadcasted_iota(jnp.int32, (1, L), 1)
    lane = jax.lax.broadcasted_iota(jnp.int32, (1, W), 1)
    bias = bias_ref[...]

    carry = jnp.zeros((1, W), jnp.float32)
    for i in range(L // c_chunk):
        vc = v_ref[0, i * c_chunk:(i + 1) * c_chunk, :]
        pc = jnp.dot(tri, vc, preferred_element_type=jnp.float32,
                     precision=jax.lax.Precision.HIGHEST) + carry
        ctx_ref[0, i * c_chunk:(i + 1) * c_chunk, :] = pc
        carry = pc[c_chunk - 1:c_chunk, :]

    Ms = []
    for h in range(hb):
        lo, hi = h * d, (h + 1) * d
        Q = q_ref[0, :, lo:hi]
        Kw = k_ref[0, :, lo:hi]
        Ks = Kw.reshape(Kw.shape[0] // 2, 2, Kw.shape[1])[:, 0, :]
        m_parts = []
        for i in range(L // q_chunk):
            qc = Q[i * q_chunk:(i + 1) * q_chunk]
            s = jax.lax.dot_general(qc, Ks, (((1,), (1,)), ((), ())),
                                    preferred_element_type=jnp.float32)
            m_parts.append(jnp.max(s, axis=1, keepdims=True)
                           - jnp.sum(s, axis=1, keepdims=True) * (1.0 / L))
        Ms.append(jnp.transpose(jnp.concatenate(m_parts, axis=0)))

    all_idxs, all_upd = [], []
    for h in range(hb):
        lo, hi = h * d, (h + 1) * d
        idxs = []
        Mw = Ms[h]
        for _ in range(u):
            mval = jnp.max(Mw)
            idx = jnp.min(jnp.where(Mw >= mval, iota, L))
            idxs.append(idx)
            Mw = jnp.where(iota == idx, _NEG_INF, Mw)
        all_idxs.append(idxs)

        Qr = jnp.concatenate(
            [q_ref[0, pl.ds(idxs[i], 1), :] for i in range(u)], axis=0)[:, lo:hi]

        K = k_ref[0, :, lo:hi]
        scores = jax.lax.dot_general(Qr, K, (((1,), (1,)), ((), ())),
                                     preferred_element_type=jnp.float32)
        scores = (scores + bias) * scale
        smax = jnp.max(scores, axis=1, keepdims=True)
        e = jnp.exp(scores - smax)
        attn = e / jnp.sum(e, axis=1, keepdims=True)
        attn_ref[0, h] = attn
        V = v_ref[0, :, lo:hi]
        update = jnp.dot(attn, V, preferred_element_type=jnp.float32)

        parts = []
        if lo > 0:
            parts.append(jnp.zeros((u, lo), jnp.float32))
        parts.append(update)
        if W > hi:
            parts.append(jnp.zeros((u, W - hi), jnp.float32))
        all_upd.append(jnp.concatenate(parts, axis=1) if len(parts) > 1
                       else update)

    for h in range(hb):
        lo, hi = h * d, (h + 1) * d
        hmask = (lane >= lo) & (lane < hi)
        idxs, upd_full = all_idxs[h], all_upd[h]
        for i in range(u):
            row = ctx_ref[0, pl.ds(idxs[i], 1), :]
            ctx_ref[0, pl.ds(idxs[i], 1), :] = jnp.where(
                hmask, upd_full[i:i + 1, :], row)


def kernel(queries, keys, values, attn_mask, rel_pos_bias_table):
    del attn_mask
    B, L, H, D = queries.shape
    u = min(_FACTOR * int(np.ceil(np.log(L))), L)
    hb = 8 if H % 8 == 0 else H

    q2 = queries.reshape(B, L, H * D)
    k2 = keys.reshape(B, L, H * D)
    v2 = values.reshape(B, L, H * D)
    idx_flat = jnp.asarray(_rpi_slice(u, L).reshape(-1))
    tab_flat = rel_pos_bias_table[:, 0]
    pad = (-tab_flat.shape[0]) % 128
    tab_flat = jnp.pad(tab_flat, (0, pad))
    bias = _bias_gather_sc(tab_flat, idx_flat).reshape(u, L)

    body = functools.partial(_body, u=u, d=D, q_chunk=min(256, L),
                             c_chunk=min(128, L), hb=hb)
    W = hb * D
    ctx, attn = pl.pallas_call(
        body,
        grid=(B, H // hb),
        in_specs=[
            pl.BlockSpec((1, L, W), lambda b, g: (b, 0, g)),
            pl.BlockSpec((1, L, W), lambda b, g: (b, 0, g)),
            pl.BlockSpec((1, L, W), lambda b, g: (b, 0, g)),
            pl.BlockSpec((u, L), lambda b, g: (0, 0)),
        ],
        out_specs=[
            pl.BlockSpec((1, L, W), lambda b, g: (b, 0, g)),
            pl.BlockSpec((1, hb, u, L), lambda b, g: (b, g, 0, 0)),
        ],
        out_shape=[
            jax.ShapeDtypeStruct((B, L, H * D), jnp.float32),
            jax.ShapeDtypeStruct((B, H, u, L), jnp.float32),
        ],
        compiler_params=pltpu.CompilerParams(
            dimension_semantics=("parallel", "parallel")),
    )(q2, k2, v2, bias)
    return ctx.reshape(B, L, H, D), attn


# speedup vs baseline: 1.9215x; 1.0200x over previous
import functools
import math

import numpy as np
import jax
import jax.numpy as jnp
from jax import lax
from jax.experimental import pallas as pl
from jax.experimental.pallas import tpu as pltpu
from jax.experimental.pallas import tpu_sc as plsc

_WS = 46
_FACTOR = 2
_NEG_INF = float("-inf")


def _rpi_slice(n_rows, n_cols):
    ws = _WS
    idx = np.arange(ws * ws)
    r, c = idx // ws, idx % ws
    ar, ac = r[:n_rows, None], c[:n_rows, None]
    br, bc = r[None, :n_cols], c[None, :n_cols]
    return ((ar - br + ws - 1) * (2 * ws - 1) + (ac - bc + ws - 1)).astype(np.int32)


def _bias_gather_sc(table_flat, idx_flat):
    n_idx = idx_flat.shape[0]
    n_tab = table_flat.shape[0]
    info = plsc.get_sparse_core_info()
    nw = info.num_cores * info.num_subcores
    lanes = info.num_lanes
    per_w = n_idx // nw
    mesh = plsc.VectorSubcoreMesh(core_axis_name="c", subcore_axis_name="s")

    @functools.partial(
        pl.kernel, mesh=mesh,
        out_type=jax.ShapeDtypeStruct((n_idx,), jnp.float32),
        compiler_params=pltpu.CompilerParams(needs_layout_passes=False),
        scratch_types=[
            pltpu.VMEM((n_tab,), jnp.float32),
            pltpu.VMEM((per_w,), jnp.int32),
            pltpu.VMEM((per_w,), jnp.float32),
        ],
    )
    def k(tab_hbm, idx_hbm, out_hbm, tab_v, idx_v, val_v):
        wid = lax.axis_index("s") * info.num_cores + lax.axis_index("c")
        base = wid * per_w
        pltpu.sync_copy(tab_hbm, tab_v)
        pltpu.sync_copy(idx_hbm.at[pl.ds(base, per_w)], idx_v)
        for j in range(per_w // lanes):
            iv = idx_v[pl.ds(j * lanes, lanes)]
            val_v[pl.ds(j * lanes, lanes)] = plsc.load_gather(tab_v, [iv])
        pltpu.sync_copy(val_v, out_hbm.at[pl.ds(base, per_w)])

    return k(table_flat, idx_flat)


def _body(q_ref, k_ref, v_ref, bias_ref, ctx_ref, attn_ref,
          *, u, d, q_chunk, c_chunk, hb):
    L = q_ref.shape[1]
    scale = 1.0 / math.sqrt(d)

    W = hb * d
    tri = (jax.lax.broadcasted_iota(jnp.int32, (c_chunk, c_chunk), 0)
           >= jax.lax.broadcasted_iota(jnp.int32, (c_chunk, c_chunk), 1)
           ).astype(jnp.float32)
    iota = jax.lax.broadcasted_iota(jnp.int32, (1, L), 1)
    lane = jax.lax.broadcasted_iota(jnp.int32, (1, W), 1)
    bias = bias_ref[...]

    carry = jnp.zeros((1, W), jnp.float32)
    for i in range(L // c_chunk):
        vc = v_ref[0, i * c_chunk:(i + 1) * c_chunk, :]
        pc = jnp.dot(tri, vc, preferred_element_type=jnp.float32,
                     precision=jax.lax.Precision.HIGHEST) + carry
        ctx_ref[0, i * c_chunk:(i + 1) * c_chunk, :] = pc
        carry = pc[c_chunk - 1:c_chunk, :]

    Ms = []
    for h in range(hb):
        lo, hi = h * d, (h + 1) * d
        Q = q_ref[0, :, lo:hi]
        Kw = k_ref[0, :, lo:hi]
        Ks = Kw.reshape(Kw.shape[0] // 2, 2, Kw.shape[1])[:, 0, :]
        m_parts = []
        for i in range(L // q_chunk):
            qc = Q[i * q_chunk:(i + 1) * q_chunk]
            s = jax.lax.dot_general(qc, Ks, (((1,), (1,)), ((), ())),
                                    preferred_element_type=jnp.float32)
            m_parts.append(jnp.max(s, axis=1, keepdims=True)
                           - jnp.sum(s, axis=1, keepdims=True) * (1.0 / L))
        Ms.append(jnp.transpose(jnp.concatenate(m_parts, axis=0)))

    all_idxs, all_upd = [], []
    for h in range(hb):
        lo, hi = h * d, (h + 1) * d
        idxs = []
        Mw = Ms[h]
        for _ in range(u):
            mval = jnp.max(Mw)
            idx = jnp.min(jnp.where(Mw >= mval, iota, L))
            idxs.append(idx)
            Mw = jnp.where(iota == idx, _NEG_INF, Mw)
        all_idxs.append(idxs)

        Qr = jnp.concatenate(
            [q_ref[0, pl.ds(idxs[i], 1), :] for i in range(u)], axis=0)[:, lo:hi]

        K = k_ref[0, :, lo:hi]
        scores = jax.lax.dot_general(Qr, K, (((1,), (1,)), ((), ())),
                                     preferred_element_type=jnp.float32)
        scores = (scores + bias) * scale
        smax = jnp.max(scores, axis=1, keepdims=True)
        e = jnp.exp(scores - smax)
        attn = e / jnp.sum(e, axis=1, keepdims=True)
        attn_ref[0, h] = attn
        V = v_ref[0, :, lo:hi]
        update = jnp.dot(attn, V, preferred_element_type=jnp.float32)

        parts = []
        if lo > 0:
            parts.append(jnp.zeros((u, lo), jnp.float32))
        parts.append(update)
        if W > hi:
            parts.append(jnp.zeros((u, W - hi), jnp.float32))
        all_upd.append(jnp.concatenate(parts, axis=1) if len(parts) > 1
                       else update)

    for h in range(hb):
        lo, hi = h * d, (h + 1) * d
        hmask = (lane >= lo) & (lane < hi)
        idxs, upd_full = all_idxs[h], all_upd[h]
        for i in range(u):
            row = ctx_ref[0, pl.ds(idxs[i], 1), :]
            ctx_ref[0, pl.ds(idxs[i], 1), :] = jnp.where(
                hmask, upd_full[i:i + 1, :], row)


def kernel(queries, keys, values, attn_mask, rel_pos_bias_table):
    del attn_mask
    B, L, H, D = queries.shape
    u = min(_FACTOR * int(np.ceil(np.log(L))), L)
    hb = 8 if H % 8 == 0 else H

    q2 = queries.reshape(B, L, H * D)
    k2 = keys.reshape(B, L, H * D)
    v2 = values.reshape(B, L, H * D)
    idx_flat = jnp.asarray(_rpi_slice(u, L).reshape(-1))
    tab_flat = rel_pos_bias_table[:, 0]
    pad = (-tab_flat.shape[0]) % 128
    tab_flat = jnp.pad(tab_flat, (0, pad))
    bias = _bias_gather_sc(tab_flat, idx_flat).reshape(u, L)

    body = functools.partial(_body, u=u, d=D, q_chunk=min(128, L),
                             c_chunk=min(128, L), hb=hb)
    W = hb * D
    ctx, attn = pl.pallas_call(
        body,
        grid=(B, H // hb),
        in_specs=[
            pl.BlockSpec((1, L, W), lambda b, g: (b, 0, g)),
            pl.BlockSpec((1, L, W), lambda b, g: (b, 0, g)),
            pl.BlockSpec((1, L, W), lambda b, g: (b, 0, g)),
            pl.BlockSpec((u, L), lambda b, g: (0, 0)),
        ],
        out_specs=[
            pl.BlockSpec((1, L, W), lambda b, g: (b, 0, g)),
            pl.BlockSpec((1, hb, u, L), lambda b, g: (b, g, 0, 0)),
        ],
        out_shape=[
            jax.ShapeDtypeStruct((B, L, H * D), jnp.float32),
            jax.ShapeDtypeStruct((B, H, u, L), jnp.float32),
        ],
        compiler_params=pltpu.CompilerParams(
            dimension_semantics=("parallel", "parallel")),
    )(q2, k2, v2, bias)
    return ctx.reshape(B, L, H, D), attn
